# Initial kernel scaffold; baseline (speedup 1.0000x reference)
import jax
import jax.numpy as jnp
from jax.experimental import pallas as pl


def kernel(seq_idx, pair_idx):
    raise NotImplementedError("write your pallas kernel here")


# speedup vs baseline: 3.2949x; 3.2949x over previous
import jax
import jax.numpy as jnp
from jax.experimental import pallas as pl
from jax.experimental.pallas import tpu as pltpu

N_BASES = 4
L = 2048
BLOCK = 256


def _body(seq_ref, pair_ref, seq_out_ref, idx_out_ref):
    i = pl.program_id(0)
    si = seq_ref[0, pl.ds(i * BLOCK, BLOCK)]
    sj = seq_ref[0, :]
    pi = pair_ref[0, pl.ds(i * BLOCK, BLOCK)]
    jj = jax.lax.broadcasted_iota(jnp.int32, (BLOCK, L), 1)
    for c in range(N_BASES):
        seq_out_ref[c] = jnp.broadcast_to(
            (si[:, None] == c).astype(jnp.float32), (BLOCK, L))
    for c in range(N_BASES):
        seq_out_ref[c + N_BASES] = jnp.broadcast_to(
            (sj[None, :] == c).astype(jnp.float32), (BLOCK, L))
    idx_out_ref[0] = (pi[:, None] == jj).astype(jnp.float32)


def kernel(seq_idx, pair_idx):
    n = seq_idx.shape[0]
    seq2d = seq_idx.reshape(1, n)
    pair2d = pair_idx.reshape(1, n)
    grid = (n // BLOCK,)
    seq_out, idx_out = pl.pallas_call(
        _body,
        grid=grid,
        in_specs=[
            pl.BlockSpec((1, n), lambda i: (0, 0)),
            pl.BlockSpec((1, n), lambda i: (0, 0)),
        ],
        out_specs=[
            pl.BlockSpec((2 * N_BASES, BLOCK, n), lambda i: (0, i, 0)),
            pl.BlockSpec((1, BLOCK, n), lambda i: (0, i, 0)),
        ],
        out_shape=[
            jax.ShapeDtypeStruct((2 * N_BASES, n, n), jnp.float32),
            jax.ShapeDtypeStruct((1, n, n), jnp.float32),
        ],
        compiler_params=pltpu.CompilerParams(
            dimension_semantics=("arbitrary",)),
    )(seq2d, pair2d)
    return (seq_out, idx_out)
